# baseline (speedup 1.0000x reference)
from itertools import permutations

import numpy as np
import jax
import jax.numpy as jnp
from jax import lax
from jax.experimental import pallas as pl
from jax.experimental.pallas import tpu as pltpu
from jax.experimental.pallas import tpu_sc as plsc

_PHI = (1.0 + 5.0**0.5) / 2.0
_C0 = _PHI / 2.0
_C1 = 0.5
_C2 = 1.0 / (2.0 * _PHI)


def _bf16_round(f):
    bits = np.float32(f).view(np.uint32)
    bits = np.uint32(bits + 0x7FFF + ((bits >> 16) & 1)) & np.uint32(0xFFFF0000)
    return float(bits.view(np.float32))


_C0B = _bf16_round(_C0)
_C2B = _bf16_round(_C2)


def _even_perms():
    def parity(p):
        par = 0
        for i in range(4):
            for j in range(i + 1, 4):
                if p[i] > p[j]:
                    par ^= 1
        return par
    return [p for p in permutations(range(4)) if parity(p) == 0]


def _weight_rows():
    rows = []
    for d in range(4):
        w = [0.0, 0.0, 0.0, 0.0]
        w[d] = 1.0
        rows.append(w)
    rows.append([0.5, 0.5, 0.5, 0.5])
    base = [_C0, _C1, _C2, 0.0]
    for p in _even_perms():
        rows.append([base[p[i]] for i in range(4)])
    return np.array(rows, dtype=np.float32)


_W_ROWS = _weight_rows()
_W_FLAT = np.zeros((72,), dtype=np.float32)
_W_FLAT[:68] = _W_ROWS.reshape(-1)

_N = 256 * 1024
_NW = 32
_PTS_PER_W = _N // _NW
_F32_PER_W = _PTS_PER_W * 4
_GROUPS = _PTS_PER_W // 16


def _sc_body(x_hbm, w_hbm, out_hbm, xv, wv, outv):
    wid = lax.axis_index("s") * 2 + lax.axis_index("c")
    base = wid * _F32_PER_W
    pltpu.sync_copy(x_hbm.at[pl.ds(base, _F32_PER_W)], xv)
    pltpu.sync_copy(w_hbm, wv)

    def step(g):
        off = (g >> 3) * 512 + (g & 7) * 16
        xs = [xv[pl.ds(off + d * 128, 16)] for d in range(4)]
        xb = []
        for xd in xs:
            bits = plsc.bitcast(xd, jnp.int32)
            bits = bits + (32767 + ((bits >> 16) & 1))
            xb.append(plsc.bitcast(bits & -65536, jnp.float32))
        a = [jnp.abs(v) for v in xb]
        neg = [v < 0.0 for v in xb]
        one = jnp.full((16,), 1.0, jnp.float32)
        sgn = [jnp.where(n, -one, one) for n in neg]
        q = [sgn[0] * (32.0 * 729.0), sgn[1] * (32.0 * 81.0),
             sgn[2] * (32.0 * 9.0), sgn[3] * 32.0]
        P = {m: [q[d] * float(m) for d in range(4)] for m in (2, 3, 4)}
        P[1] = q
        S = [[c * ad for ad in a] for c in (_C0B, 0.5, _C2B)]

        scores = []
        keys = []
        for d in range(4):
            scores.append(a[d])
            keys.append(P[4][d] + float(d))
        sh = ((S[1][0] + S[1][1]) + S[1][2]) + S[1][3]
        scores.append(sh)
        keys.append((((P[2][0] + P[2][1]) + P[2][2]) + P[2][3]) + 4.0)
        pair_cache = {}
        def pair(c_i, i, c_j, j, M, tag):
            k = (tag, c_i, i, c_j, j)
            if k not in pair_cache:
                pair_cache[k] = (M[c_i][i] + M[c_j][j])
            return pair_cache[k]
        cand = 5
        for p in _even_perms():
            nz = [i for i in range(4) if p[i] < 3]
            i0_, i1_, i2_ = nz
            s = pair(p[i0_], i0_, p[i1_], i1_, S, "s") + S[p[i2_]][i2_]
            key = (pair(3 - p[i0_], i0_, 3 - p[i1_], i1_, P, "k")
                   + P[3 - p[i2_]][i2_]) + float(cand)
            scores.append(s)
            keys.append(key)
            cand += 1

        t = scores
        while len(t) > 1:
            t = [jnp.maximum(t[i], t[i + 1]) for i in range(0, len(t) - 1, 2)] \
                + ([t[-1]] if len(t) & 1 else [])
        best = t[0]
        big = jnp.full((16,), 1e9, jnp.float32)
        t = [jnp.where(s == best, k, big) for s, k in zip(scores, keys)]
        while len(t) > 1:
            t = [jnp.minimum(t[i], t[i + 1]) for i in range(0, len(t) - 1, 2)] \
                + ([t[-1]] if len(t) & 1 else [])
        bidx = t[0].astype(jnp.int32) & 31

        widx = bidx * 4
        for d in range(4):
            w = plsc.load_gather(wv, [widx + d])
            outv[pl.ds(off + d * 128, 16)] = w * sgn[d]

    plsc.parallel_loop(0, _GROUPS, unroll=4)(step)
    pltpu.sync_copy(outv, out_hbm.at[pl.ds(base, _F32_PER_W)])


def _quantize_flat(xf, wf):
    mesh = plsc.VectorSubcoreMesh(core_axis_name="c", subcore_axis_name="s")
    return pl.kernel(
        _sc_body,
        out_type=jax.ShapeDtypeStruct((_N * 4,), jnp.float32),
        mesh=mesh,
        scratch_types=[
            pltpu.VMEM((_F32_PER_W,), jnp.float32),
            pltpu.VMEM((72,), jnp.float32),
            pltpu.VMEM((_F32_PER_W,), jnp.float32),
        ],
        compiler_params=pltpu.CompilerParams(
            use_tc_tiling_on_sc=False, needs_layout_passes=False
        ),
    )(xf, wf)


def kernel(x, vertices):
    del vertices
    xl = x.reshape(256, 8, 128, 4).transpose(0, 1, 3, 2).reshape(-1)
    out = _quantize_flat(xl, jnp.asarray(_W_FLAT))
    return (out.reshape(256, 8, 4, 128)
               .transpose(0, 1, 3, 2)
               .reshape(x.shape))


# speedup vs baseline: 1.7714x; 1.7714x over previous
from itertools import permutations

import numpy as np
import jax
import jax.numpy as jnp
from jax import lax
from jax.experimental import pallas as pl
from jax.experimental.pallas import tpu as pltpu
from jax.experimental.pallas import tpu_sc as plsc

_PHI = (1.0 + 5.0**0.5) / 2.0
_C0 = _PHI / 2.0
_C1 = 0.5
_C2 = 1.0 / (2.0 * _PHI)


def _bf16_round(f):
    bits = np.float32(f).view(np.uint32)
    bits = np.uint32(bits + 0x7FFF + ((bits >> 16) & 1)) & np.uint32(0xFFFF0000)
    return float(bits.view(np.float32))


_C0B = _bf16_round(_C0)
_C2B = _bf16_round(_C2)


def _even_perms():
    def parity(p):
        par = 0
        for i in range(4):
            for j in range(i + 1, 4):
                if p[i] > p[j]:
                    par ^= 1
        return par
    return [p for p in permutations(range(4)) if parity(p) == 0]


def _weight_rows():
    rows = []
    for d in range(4):
        w = [0.0, 0.0, 0.0, 0.0]
        w[d] = 1.0
        rows.append(w)
    rows.append([0.5, 0.5, 0.5, 0.5])
    base = [_C0, _C1, _C2, 0.0]
    for p in _even_perms():
        rows.append([base[p[i]] for i in range(4)])
    return np.array(rows, dtype=np.float32)


_W_ROWS = _weight_rows()
_W_FLAT = np.zeros((72,), dtype=np.float32)
_W_FLAT[:68] = _W_ROWS.reshape(-1)

_N = 256 * 1024
_NW = 32
_PTS_PER_W = _N // _NW
_F32_PER_W = _PTS_PER_W * 4
_GROUPS = _PTS_PER_W // 16


def _sc_body(x_hbm, w_hbm, out_hbm, xv, wv, outv):
    wid = lax.axis_index("s") * 2 + lax.axis_index("c")
    base = wid * _F32_PER_W
    pltpu.sync_copy(x_hbm.at[pl.ds(base, _F32_PER_W)], xv)
    pltpu.sync_copy(w_hbm, wv)

    def step(g):
        off = (g >> 3) * 512 + (g & 7) * 16
        xs = [xv[pl.ds(off + d * 128, 16)] for d in range(4)]
        xb = []
        for xd in xs:
            bits = plsc.bitcast(xd, jnp.int32)
            bits = bits + (32767 + ((bits >> 16) & 1))
            xb.append(plsc.bitcast(bits & -65536, jnp.float32))
        a = [jnp.abs(v) for v in xb]
        neg = [v < 0.0 for v in xb]
        one = jnp.full((16,), 1.0, jnp.float32)
        sgn = [jnp.where(n, -one, one) for n in neg]
        q = [sgn[0] * (32.0 * 729.0), sgn[1] * (32.0 * 81.0),
             sgn[2] * (32.0 * 9.0), sgn[3] * 32.0]
        P = {m: [q[d] * float(m) for d in range(4)] for m in (2, 3, 4)}
        P[1] = q
        S = [[c * ad for ad in a] for c in (_C0B, 0.5, _C2B)]

        scores = []
        keys = []
        for d in range(4):
            scores.append(a[d])
            keys.append(P[4][d] + float(d))
        sh = ((S[1][0] + S[1][1]) + S[1][2]) + S[1][3]
        scores.append(sh)
        keys.append((((P[2][0] + P[2][1]) + P[2][2]) + P[2][3]) + 4.0)
        pair_cache = {}
        def pair(c_i, i, c_j, j, M, tag):
            k = (tag, c_i, i, c_j, j)
            if k not in pair_cache:
                pair_cache[k] = (M[c_i][i] + M[c_j][j])
            return pair_cache[k]
        cand = 5
        for p in _even_perms():
            nz = [i for i in range(4) if p[i] < 3]
            i0_, i1_, i2_ = nz
            s = pair(p[i0_], i0_, p[i1_], i1_, S, "s") + S[p[i2_]][i2_]
            key = (pair(3 - p[i0_], i0_, 3 - p[i1_], i1_, P, "k")
                   + P[3 - p[i2_]][i2_]) + float(cand)
            scores.append(s)
            keys.append(key)
            cand += 1

        t = scores
        while len(t) > 1:
            t = [jnp.maximum(t[i], t[i + 1]) for i in range(0, len(t) - 1, 2)] \
                + ([t[-1]] if len(t) & 1 else [])
        best = t[0]
        big = jnp.full((16,), 1e9, jnp.float32)
        t = [jnp.where(s == best, k, big) for s, k in zip(scores, keys)]
        while len(t) > 1:
            t = [jnp.minimum(t[i], t[i + 1]) for i in range(0, len(t) - 1, 2)] \
                + ([t[-1]] if len(t) & 1 else [])
        bidx = t[0].astype(jnp.int32) & 31

        widx = bidx * 4
        for d in range(4):
            w = plsc.load_gather(wv, [widx + d])
            outv[pl.ds(off + d * 128, 16)] = w * sgn[d]

    plsc.parallel_loop(0, _GROUPS, unroll=2)(step)
    pltpu.sync_copy(outv, out_hbm.at[pl.ds(base, _F32_PER_W)])


def _quantize_flat(xf, wf):
    mesh = plsc.VectorSubcoreMesh(core_axis_name="c", subcore_axis_name="s")
    return pl.kernel(
        _sc_body,
        out_type=jax.ShapeDtypeStruct((_N * 4,), jnp.float32),
        mesh=mesh,
        scratch_types=[
            pltpu.VMEM((_F32_PER_W,), jnp.float32),
            pltpu.VMEM((72,), jnp.float32),
            pltpu.VMEM((_F32_PER_W,), jnp.float32),
        ],
        compiler_params=pltpu.CompilerParams(
            use_tc_tiling_on_sc=False, needs_layout_passes=False
        ),
    )(xf, wf)


def kernel(x, vertices):
    del vertices
    xl = x.reshape(256, 8, 128, 4).transpose(0, 1, 3, 2).reshape(-1)
    out = _quantize_flat(xl, jnp.asarray(_W_FLAT))
    return (out.reshape(256, 8, 4, 128)
               .transpose(0, 1, 3, 2)
               .reshape(x.shape))


# speedup vs baseline: 1.8777x; 1.0600x over previous
from itertools import permutations

import numpy as np
import jax
import jax.numpy as jnp
from jax import lax
from jax.experimental import pallas as pl
from jax.experimental.pallas import tpu as pltpu
from jax.experimental.pallas import tpu_sc as plsc

_PHI = (1.0 + 5.0**0.5) / 2.0
_C0 = _PHI / 2.0
_C1 = 0.5
_C2 = 1.0 / (2.0 * _PHI)


def _bf16_round(f):
    bits = np.float32(f).view(np.uint32)
    bits = np.uint32(bits + 0x7FFF + ((bits >> 16) & 1)) & np.uint32(0xFFFF0000)
    return float(bits.view(np.float32))


_C0B = _bf16_round(_C0)
_C2B = _bf16_round(_C2)


def _even_perms():
    def parity(p):
        par = 0
        for i in range(4):
            for j in range(i + 1, 4):
                if p[i] > p[j]:
                    par ^= 1
        return par
    return [p for p in permutations(range(4)) if parity(p) == 0]


def _weight_rows():
    rows = []
    for d in range(4):
        w = [0.0, 0.0, 0.0, 0.0]
        w[d] = 1.0
        rows.append(w)
    rows.append([0.5, 0.5, 0.5, 0.5])
    base = [_C0, _C1, _C2, 0.0]
    for p in _even_perms():
        rows.append([base[p[i]] for i in range(4)])
    return np.array(rows, dtype=np.float32)


_W_ROWS = _weight_rows()
_W_FLAT = np.zeros((80,), dtype=np.float32)
_W_FLAT[:68] = _W_ROWS.reshape(-1)

_N = 256 * 1024
_NW = 32
_PTS_PER_W = _N // _NW
_F32_PER_W = _PTS_PER_W * 4
_GROUPS = _PTS_PER_W // 16


def _sc_body(x_hbm, out_hbm, xv, wv, outv, lut, sem_a, sem_b, sem_o):
    wid = lax.axis_index("s") * 2 + lax.axis_index("c")
    base = wid * _F32_PER_W
    half = _F32_PER_W // 2
    cp_a = pltpu.async_copy(x_hbm.at[pl.ds(base, half)],
                            xv.at[pl.ds(0, half)], sem_a)
    cp_b = pltpu.async_copy(x_hbm.at[pl.ds(base + half, half)],
                            xv.at[pl.ds(half, half)], sem_b)
    lane = jnp.arange(16, dtype=jnp.int32)
    for c in range(5):
        chunk = _W_FLAT[c * 16:(c + 1) * 16]
        acc = jnp.zeros((16,), jnp.float32)
        for v in sorted(set(float(t) for t in chunk) - {0.0}):
            mask = 0
            for i in range(16):
                if float(chunk[i]) == v:
                    mask |= 1 << i
            bit = (mask >> lane) & 1
            acc = acc + v * bit.astype(jnp.float32)
        wv[pl.ds(c * 16, 16)] = acc
    lane_sgn = [(1 - 2 * ((lane >> d) & 1)).astype(jnp.float32) for d in range(4)]
    base96 = [_C0, _C1, _C2, 0.0]
    ranks_list = []
    for d in range(4):
        r = [0, 0, 0, 0]
        r[d] = 4
        ranks_list.append(r)
    ranks_list.append([2, 2, 2, 2])
    for p in _even_perms():
        ranks_list.append([{_C0: 3, _C1: 2, _C2: 1, 0.0: 0}[base96[p[i]]]
                           for i in range(4)])
    for ci, ranks in enumerate(ranks_list):
        key = jnp.zeros((16,), jnp.float32)
        for d in range(4):
            if ranks[d]:
                key = key + lane_sgn[d] * float(ranks[d] * 9 ** (3 - d))
        kidx = key.astype(jnp.int32) + 3280
        plsc.store_scatter(lut, [kidx], jnp.full((16,), ci * 4, jnp.int32))

    def step(g):
        off = (g >> 3) * 512 + (g & 7) * 16
        xs = [xv[pl.ds(off + d * 128, 16)] for d in range(4)]
        xb = []
        for xd in xs:
            bits = plsc.bitcast(xd, jnp.int32)
            bits = bits + (32767 + ((bits >> 16) & 1))
            xb.append(plsc.bitcast(bits & -65536, jnp.float32))
        a = [jnp.abs(v) for v in xb]
        neg = [v < 0.0 for v in xb]
        one = jnp.full((16,), 1.0, jnp.float32)
        sgn = [jnp.where(n, -one, one) for n in neg]
        q = [sgn[0] * 729.0, sgn[1] * 81.0, sgn[2] * 9.0, sgn[3]]
        P = {m: [q[d] * float(m) for d in range(4)] for m in (2, 3, 4)}
        P[1] = q
        S = [[c * ad for ad in a] for c in (_C0B, 0.5, _C2B)]

        scores = []
        keys = []
        for d in range(4):
            scores.append(a[d])
            keys.append(P[4][d])
        sh = ((S[1][0] + S[1][1]) + S[1][2]) + S[1][3]
        scores.append(sh)
        keys.append(((P[2][0] + P[2][1]) + P[2][2]) + P[2][3])
        pair_cache = {}
        def pair(c_i, i, c_j, j, M, tag):
            k = (tag, c_i, i, c_j, j)
            if k not in pair_cache:
                pair_cache[k] = (M[c_i][i] + M[c_j][j])
            return pair_cache[k]
        for p in _even_perms():
            nz = [i for i in range(4) if p[i] < 3]
            i0_, i1_, i2_ = nz
            s = pair(p[i0_], i0_, p[i1_], i1_, S, "s") + S[p[i2_]][i2_]
            key = (pair(3 - p[i0_], i0_, 3 - p[i1_], i1_, P, "k")
                   + P[3 - p[i2_]][i2_])
            scores.append(s)
            keys.append(key)

        t = scores
        while len(t) > 1:
            t = [jnp.maximum(t[i], t[i + 1]) for i in range(0, len(t) - 1, 2)] \
                + ([t[-1]] if len(t) & 1 else [])
        best = t[0]
        big = jnp.full((16,), 1e9, jnp.float32)
        t = [jnp.where(s == best, k, big) for s, k in zip(scores, keys)]
        while len(t) > 1:
            t = [jnp.minimum(t[i], t[i + 1]) for i in range(0, len(t) - 1, 2)] \
                + ([t[-1]] if len(t) & 1 else [])
        widx = plsc.load_gather(lut, [t[0].astype(jnp.int32) + 3280])
        for d in range(4):
            w = plsc.load_gather(wv, [widx + d])
            outv[pl.ds(off + d * 128, 16)] = w * sgn[d]

    cp_a.wait()
    plsc.parallel_loop(0, _GROUPS // 2, unroll=1)(step)
    cp_o = pltpu.async_copy(outv.at[pl.ds(0, half)],
                            out_hbm.at[pl.ds(base, half)], sem_o)
    cp_b.wait()
    plsc.parallel_loop(_GROUPS // 2, _GROUPS, unroll=1)(step)
    cp_o.wait()
    pltpu.sync_copy(outv.at[pl.ds(half, half)],
                    out_hbm.at[pl.ds(base + half, half)])


def _quantize_flat(xf):
    mesh = plsc.VectorSubcoreMesh(core_axis_name="c", subcore_axis_name="s")
    return pl.kernel(
        _sc_body,
        out_type=jax.ShapeDtypeStruct((_N * 4,), jnp.float32),
        mesh=mesh,
        scratch_types=[
            pltpu.VMEM((_F32_PER_W,), jnp.float32),
            pltpu.VMEM((80,), jnp.float32),
            pltpu.VMEM((_F32_PER_W,), jnp.float32),
            pltpu.VMEM((6568,), jnp.int32),
            pltpu.SemaphoreType.DMA,
            pltpu.SemaphoreType.DMA,
            pltpu.SemaphoreType.DMA,
        ],
        compiler_params=pltpu.CompilerParams(
            use_tc_tiling_on_sc=False, needs_layout_passes=False
        ),
    )(xf)


def kernel(x, vertices):
    del vertices
    xl = x.reshape(256, 8, 128, 4).transpose(0, 1, 3, 2).reshape(-1)
    out = _quantize_flat(xl)
    return (out.reshape(256, 8, 4, 128)
               .transpose(0, 1, 3, 2)
               .reshape(x.shape))


# speedup vs baseline: 1.9070x; 1.0156x over previous
from itertools import permutations

import numpy as np
import jax
import jax.numpy as jnp
from jax import lax
from jax.experimental import pallas as pl
from jax.experimental.pallas import tpu as pltpu
from jax.experimental.pallas import tpu_sc as plsc

_PHI = (1.0 + 5.0**0.5) / 2.0
_C0 = _PHI / 2.0
_C1 = 0.5
_C2 = 1.0 / (2.0 * _PHI)


def _bf16_round(f):
    bits = np.float32(f).view(np.uint32)
    bits = np.uint32(bits + 0x7FFF + ((bits >> 16) & 1)) & np.uint32(0xFFFF0000)
    return float(bits.view(np.float32))


_C0B = _bf16_round(_C0)
_C2B = _bf16_round(_C2)


def _even_perms():
    def parity(p):
        par = 0
        for i in range(4):
            for j in range(i + 1, 4):
                if p[i] > p[j]:
                    par ^= 1
        return par
    return [p for p in permutations(range(4)) if parity(p) == 0]


def _weight_rows():
    rows = []
    for d in range(4):
        w = [0.0, 0.0, 0.0, 0.0]
        w[d] = 1.0
        rows.append(w)
    rows.append([0.5, 0.5, 0.5, 0.5])
    base = [_C0, _C1, _C2, 0.0]
    for p in _even_perms():
        rows.append([base[p[i]] for i in range(4)])
    return np.array(rows, dtype=np.float32)


_W_ROWS = _weight_rows()
_W_FLAT = np.zeros((80,), dtype=np.float32)
_W_FLAT[:68] = _W_ROWS.reshape(-1)

_N = 256 * 1024
_NW = 32
_PTS_PER_W = _N // _NW
_F32_PER_W = _PTS_PER_W * 4
_GROUPS = _PTS_PER_W // 16


def _sc_body(x_hbm, out_hbm, xv, wv, outv, lut, sem_a, sem_b, sem_o):
    wid = lax.axis_index("s") * 2 + lax.axis_index("c")
    base = wid * _F32_PER_W
    half = _F32_PER_W // 2
    cp_a = pltpu.async_copy(x_hbm.at[pl.ds(base, half)],
                            xv.at[pl.ds(0, half)], sem_a)
    cp_b = pltpu.async_copy(x_hbm.at[pl.ds(base + half, half)],
                            xv.at[pl.ds(half, half)], sem_b)
    lane = jnp.arange(16, dtype=jnp.int32)
    for c in range(5):
        chunk = _W_FLAT[c * 16:(c + 1) * 16]
        acc = jnp.zeros((16,), jnp.float32)
        for v in sorted(set(float(t) for t in chunk) - {0.0}):
            mask = 0
            for i in range(16):
                if float(chunk[i]) == v:
                    mask |= 1 << i
            bit = (mask >> lane) & 1
            acc = acc + v * bit.astype(jnp.float32)
        wv[pl.ds(c * 16, 16)] = acc
    lane_sgn = [(1 - 2 * ((lane >> d) & 1)).astype(jnp.float32) for d in range(4)]
    base96 = [_C0, _C1, _C2, 0.0]
    ranks_list = []
    for d in range(4):
        r = [0, 0, 0, 0]
        r[d] = 4
        ranks_list.append(r)
    ranks_list.append([2, 2, 2, 2])
    for p in _even_perms():
        ranks_list.append([{_C0: 3, _C1: 2, _C2: 1, 0.0: 0}[base96[p[i]]]
                           for i in range(4)])
    for ci, ranks in enumerate(ranks_list):
        key = jnp.zeros((16,), jnp.float32)
        for d in range(4):
            if ranks[d]:
                key = key + lane_sgn[d] * float(ranks[d] * 9 ** (3 - d))
        kidx = key.astype(jnp.int32) + 3280
        plsc.store_scatter(lut, [kidx], jnp.full((16,), ci * 4, jnp.int32))

    def step(g):
        off = (g >> 3) * 512 + (g & 7) * 16
        xs = [xv[pl.ds(off + d * 128, 16)] for d in range(4)]
        xb = []
        for xd in xs:
            g = xd * 65537.0
            xb.append(g + (xd - g))
        a = [jnp.abs(v) for v in xb]
        neg = [v < 0.0 for v in xb]
        one = jnp.full((16,), 1.0, jnp.float32)
        sgn = [jnp.where(n, -one, one) for n in neg]
        q = [sgn[0] * 729.0, sgn[1] * 81.0, sgn[2] * 9.0, sgn[3]]
        P = {m: [q[d] * float(m) for d in range(4)] for m in (2, 3, 4)}
        P[1] = q
        S = [[c * ad for ad in a] for c in (_C0B, 0.5, _C2B)]

        scores = []
        keys = []
        for d in range(4):
            scores.append(a[d])
            keys.append(P[4][d])
        sh = ((S[1][0] + S[1][1]) + S[1][2]) + S[1][3]
        scores.append(sh)
        keys.append(((P[2][0] + P[2][1]) + P[2][2]) + P[2][3])
        pair_cache = {}
        def pair(c_i, i, c_j, j, M, tag):
            k = (tag, c_i, i, c_j, j)
            if k not in pair_cache:
                pair_cache[k] = (M[c_i][i] + M[c_j][j])
            return pair_cache[k]
        for p in _even_perms():
            nz = [i for i in range(4) if p[i] < 3]
            i0_, i1_, i2_ = nz
            s = pair(p[i0_], i0_, p[i1_], i1_, S, "s") + S[p[i2_]][i2_]
            key = (pair(3 - p[i0_], i0_, 3 - p[i1_], i1_, P, "k")
                   + P[3 - p[i2_]][i2_])
            scores.append(s)
            keys.append(key)

        t = scores
        while len(t) > 1:
            t = [jnp.maximum(t[i], t[i + 1]) for i in range(0, len(t) - 1, 2)] \
                + ([t[-1]] if len(t) & 1 else [])
        best = t[0]
        big = jnp.full((16,), 1e9, jnp.float32)
        t = [jnp.where(s == best, k, big) for s, k in zip(scores, keys)]
        while len(t) > 1:
            t = [jnp.minimum(t[i], t[i + 1]) for i in range(0, len(t) - 1, 2)] \
                + ([t[-1]] if len(t) & 1 else [])
        widx = plsc.load_gather(lut, [t[0].astype(jnp.int32) + 3280])
        for d in range(4):
            w = plsc.load_gather(wv, [widx + d])
            outv[pl.ds(off + d * 128, 16)] = w * sgn[d]

    cp_a.wait()
    plsc.parallel_loop(0, _GROUPS // 2, unroll=1)(step)
    cp_o = pltpu.async_copy(outv.at[pl.ds(0, half)],
                            out_hbm.at[pl.ds(base, half)], sem_o)
    cp_b.wait()
    plsc.parallel_loop(_GROUPS // 2, _GROUPS, unroll=1)(step)
    cp_o.wait()
    pltpu.sync_copy(outv.at[pl.ds(half, half)],
                    out_hbm.at[pl.ds(base + half, half)])


def _quantize_flat(xf):
    mesh = plsc.VectorSubcoreMesh(core_axis_name="c", subcore_axis_name="s")
    return pl.kernel(
        _sc_body,
        out_type=jax.ShapeDtypeStruct((_N * 4,), jnp.float32),
        mesh=mesh,
        scratch_types=[
            pltpu.VMEM((_F32_PER_W,), jnp.float32),
            pltpu.VMEM((80,), jnp.float32),
            pltpu.VMEM((_F32_PER_W,), jnp.float32),
            pltpu.VMEM((6568,), jnp.int32),
            pltpu.SemaphoreType.DMA,
            pltpu.SemaphoreType.DMA,
            pltpu.SemaphoreType.DMA,
        ],
        compiler_params=pltpu.CompilerParams(
            use_tc_tiling_on_sc=False, needs_layout_passes=False
        ),
    )(xf)


def kernel(x, vertices):
    del vertices
    xl = x.reshape(256, 8, 128, 4).transpose(0, 1, 3, 2).reshape(-1)
    out = _quantize_flat(xl)
    return (out.reshape(256, 8, 4, 128)
               .transpose(0, 1, 3, 2)
               .reshape(x.shape))
